# baseline (speedup 1.0000x reference)
import jax
import jax.numpy as jnp
from jax import lax
from jax.experimental import pallas as pl
from jax.experimental.pallas import tpu as pltpu
from jax.experimental.pallas import tpu_sc as plsc

B, N, C, H, W, R = 4, 8192, 64, 128, 128, 8
BN = B * N
NW = 32
TPB = NW // B
CPT = C // TPB
PLANE = H * W * R
HALF = PLANE // 2
SEG = N // TPB


def _sc_body(p3_h, obs_h, egm_h, out_h, *,
             p3c, idx00_v, fx_v, fy_v, fa_v, plane_v, m_v, d2_v, obs_v):
    cid = lax.axis_index("c")
    sid = lax.axis_index("s")
    wid = cid * 16 + sid
    b = wid // TPB
    k = wid % TPB

    pltpu.sync_copy(obs_h.at[pl.ds(b * C, C)], obs_v)

    @pl.loop(0, N // 256)
    def _meta(ch):
        pltpu.sync_copy(p3_h.at[pl.ds((b * N + ch * 256) * 3, 768)], p3c)

        @pl.loop(0, 16)
        def _g(g2):
            lane3 = lax.iota(jnp.int32, 16) * 3 + g2 * 48
            x = plsc.load_gather(p3c, [lane3])
            y = plsc.load_gather(p3c, [lane3 + 1])
            ang = plsc.load_gather(p3c, [lane3 + 2])
            t = ang * (1.0 / 360.0)
            t = t - t.astype(jnp.int32).astype(jnp.float32)
            a = t * 8.0
            x0 = jnp.minimum(x.astype(jnp.int32), W - 2)
            y0 = jnp.minimum(y.astype(jnp.int32), H - 2)
            a0 = jnp.minimum(a.astype(jnp.int32), R - 1)
            sl = pl.ds(ch * 256 + g2 * 16, 16)
            idx00_v[sl] = (y0 * W + x0) * R + a0
            fx_v[sl] = x - x0.astype(jnp.float32)
            fy_v[sl] = y - y0.astype(jnp.float32)
            fa_v[sl] = a - a0.astype(jnp.float32)
            d2_v[sl] = jnp.zeros((16,), jnp.float32)

    @pl.loop(0, CPT)
    def _plane(ci):
        ck = k * CPT + ci
        base = (b * C + ck) * PLANE

        def do_pass(h, final):
            @pl.loop(0, N // 16)
            def _grp(g):
                sl = pl.ds(g * 16, 16)
                i00 = idx00_v[sl]
                fx = fx_v[sl]
                fy = fy_v[sl]
                fa = fa_v[sl]
                gx = 1.0 - fx
                gy = 1.0 - fy
                ga = 1.0 - fa
                a0 = i00 & 7
                da1 = jnp.where(a0 == 7, -7, 1)
                loc = i00 - h * HALF
                m = None
                for dy in (0, 1):
                    wy = fy if dy else gy
                    for dx in (0, 1):
                        wxy = wy * (fx if dx else gx)
                        cbase = loc + (dy * W * R + dx * R)
                        for da in (0, 1):
                            cidx = cbase + da1 if da else cbase
                            msk = (cidx >= 0) & (cidx < HALF)
                            v = plsc.load_gather(plane_v, [cidx], mask=msk)
                            v = jnp.where(msk, v, 0.0)
                            wgt = wxy * (fa if da else ga)
                            m = v * wgt if m is None else m + v * wgt
                if final:
                    mm = m_v[sl] + m
                    o = plsc.load_gather(obs_v, [jnp.full((16,), ck,
                                                          jnp.int32)])
                    u = mm - o
                    d2_v[sl] = d2_v[sl] + u * u
                else:
                    m_v[sl] = m

        pltpu.sync_copy(egm_h.at[pl.ds(base, HALF)], plane_v)
        do_pass(0, False)
        pltpu.sync_copy(egm_h.at[pl.ds(base + HALF, HALF)], plane_v)
        do_pass(1, True)

    pltpu.sync_copy(d2_v, out_h.at[pl.ds((k * B + b) * N, N)])


def _sc_weights(p3, obs_flat, egm_flat):
    mesh = plsc.VectorSubcoreMesh(core_axis_name="c", subcore_axis_name="s",
                                  num_cores=2, num_subcores=16)
    fn = pl.kernel(
        _sc_body,
        out_type=jax.ShapeDtypeStruct((TPB * BN,), jnp.float32),
        mesh=mesh,
        scratch_types=dict(
            p3c=pltpu.VMEM((768,), jnp.float32),
            idx00_v=pltpu.VMEM((N,), jnp.int32),
            fx_v=pltpu.VMEM((N,), jnp.float32),
            fy_v=pltpu.VMEM((N,), jnp.float32),
            fa_v=pltpu.VMEM((N,), jnp.float32),
            plane_v=pltpu.VMEM((HALF,), jnp.float32),
            m_v=pltpu.VMEM((N,), jnp.float32),
            d2_v=pltpu.VMEM((N,), jnp.float32),
            obs_v=pltpu.VMEM((C,), jnp.float32),
        ),
        compiler_params=pltpu.CompilerParams(needs_layout_passes=False,
                                             use_tc_tiling_on_sc=False),
    )
    return fn(p3, obs_flat, egm_flat)


def _norm_body(d2p_ref, lw_ref, o_ref):
    d2 = jnp.sum(d2p_ref[...], axis=0)
    w = jnp.exp(lw_ref[...] - d2 * (1.0 / 128.0))
    o_ref[...] = w / jnp.sum(w, axis=1, keepdims=True)


def kernel(particles, encoded_global_map, encoded_observations,
           unnormalized_resampled_particle_log_weights):
    d2p = _sc_weights(
        particles.reshape(BN * 3),
        encoded_observations.reshape(B * C),
        encoded_global_map.reshape(B * C * PLANE)).reshape(TPB, B, N)
    return pl.pallas_call(
        _norm_body,
        out_shape=jax.ShapeDtypeStruct((B, N), jnp.float32),
    )(d2p, unnormalized_resampled_particle_log_weights)


# speedup vs baseline: 3.2228x; 3.2228x over previous
import jax
import jax.numpy as jnp
from jax import lax
from jax.experimental import pallas as pl
from jax.experimental.pallas import tpu as pltpu
from jax.experimental.pallas import tpu_sc as plsc

B, N, C, H, W, R = 4, 8192, 64, 128, 128, 8
BN = B * N
NW = 32
P_TILE = BN // NW
CH = 64
NCH = P_TILE // CH
ROWS_PER_BATCH = H * W * R


C2 = C // 2


def _sc_body(xs_h, ys_h, as_h, lw_h, obs_h, table_h, out_h, *,
             x_v, y_v, a_v, lw_v, obs_v, w_v, idx_refs, wg_refs,
             rows_a, rows_b, sem_a, sem_b):
    cid = lax.axis_index("c")
    sid = lax.axis_index("s")
    wid = sid * 2 + cid
    b = wid // (NW // B)
    base_row = b * ROWS_PER_BATCH
    pstart = wid * P_TILE

    pltpu.sync_copy(obs_h.at[pl.ds(b * C, C)], obs_v)
    pltpu.sync_copy(xs_h.at[pl.ds(pstart, P_TILE)], x_v)
    pltpu.sync_copy(ys_h.at[pl.ds(pstart, P_TILE)], y_v)
    pltpu.sync_copy(as_h.at[pl.ds(pstart, P_TILE)], a_v)
    pltpu.sync_copy(lw_h.at[pl.ds(pstart, P_TILE)], lw_v)

    @pl.loop(0, P_TILE // 16)
    def _grp(g):
        sl = pl.ds(g * 16, 16)
        x = x_v[sl]
        y = y_v[sl]
        ang = a_v[sl]
        t = ang * (1.0 / 360.0)
        t = t - t.astype(jnp.int32).astype(jnp.float32)
        a = t * 8.0
        x0 = jnp.minimum(x.astype(jnp.int32), W - 2)
        y0 = jnp.minimum(y.astype(jnp.int32), H - 2)
        a0 = jnp.minimum(a.astype(jnp.int32), R - 1)
        fx = x - x0.astype(jnp.float32)
        fy = y - y0.astype(jnp.float32)
        fa = a - a0.astype(jnp.float32)
        gx = 1.0 - fx
        gy = 1.0 - fy
        ga = 1.0 - fa
        a1 = a0 + 1
        a1 = jnp.where(a1 == R, 0, a1)
        r00 = base_row + (y0 * W + x0) * R
        r01 = r00 + R
        r10 = r00 + W * R
        r11 = r10 + R
        off = (g // 4) * (2 * CH) + (g % 4) * 16
        sl0 = pl.ds(off, 16)
        sl1 = pl.ds(off + CH, 16)
        idx_refs[0][sl0] = r00 + a0
        idx_refs[0][sl1] = r00 + a1
        idx_refs[1][sl0] = r01 + a0
        idx_refs[1][sl1] = r01 + a1
        idx_refs[2][sl0] = r10 + a0
        idx_refs[2][sl1] = r10 + a1
        idx_refs[3][sl0] = r11 + a0
        idx_refs[3][sl1] = r11 + a1
        wg_refs[0][sl] = gy * gx * ga
        wg_refs[1][sl] = gy * gx * fa
        wg_refs[2][sl] = gy * fx * ga
        wg_refs[3][sl] = gy * fx * fa
        wg_refs[4][sl] = fy * gx * ga
        wg_refs[5][sl] = fy * gx * fa
        wg_refs[6][sl] = fy * fx * ga
        wg_refs[7][sl] = fy * fx * fa

    def fire(rows, sem, ci):
        for p in range(4):
            pltpu.async_copy(
                table_h.at[idx_refs[p].at[pl.ds(ci * 2 * CH, 2 * CH)]],
                rows[p], sem)

    def drain(rows, sem):
        for p in range(4):
            pltpu.make_async_copy(table_h.at[pl.ds(0, 2 * CH)], rows[p], sem
                                  ).wait()

    def compute(rows, ci):
        @pl.loop(0, CH // 16)
        def _grp2(g):
            gsl = pl.ds(ci * CH + g * 16, 16)
            pidx = lax.iota(jnp.int32, 16) + g * 16
            ws = [wg_refs[j][gsl] for j in range(8)]

            def cbody(cc, d2):
                cs = jnp.full((16,), cc, jnp.int32)
                m = None
                for j in range(8):
                    p, jj = j // 2, j % 2
                    v = plsc.load_gather(rows[p], [pidx + jj * CH, cs])
                    m = v * ws[j] if m is None else m + v * ws[j]
                o = plsc.load_gather(obs_v, [cs])
                u = m - o
                return d2 + u * u

            d2 = lax.fori_loop(0, C, cbody, jnp.zeros((16,), jnp.float32),
                               unroll=4)
            w_v[gsl] = jnp.exp(lw_v[gsl] - d2 * (1.0 / 128.0))

    fire(rows_a, sem_a, 0)

    @pl.loop(0, NCH // 2)
    def _pipe(i):
        ci0 = i * 2
        fire(rows_b, sem_b, ci0 + 1)
        drain(rows_a, sem_a)
        compute(rows_a, ci0)

        @pl.when(i < NCH // 2 - 1)
        def _():
            fire(rows_a, sem_a, ci0 + 2)

        drain(rows_b, sem_b)
        compute(rows_b, ci0 + 1)

    pltpu.sync_copy(w_v, out_h.at[pl.ds(pstart, P_TILE)])


def _sc_weights(xs, ys, angs, lws, obs_flat, table):
    mesh = plsc.VectorSubcoreMesh(core_axis_name="c", subcore_axis_name="s",
                                  num_cores=2, num_subcores=16)
    fn = pl.kernel(
        _sc_body,
        out_type=jax.ShapeDtypeStruct((BN,), jnp.float32),
        mesh=mesh,
        scratch_types=dict(
            x_v=pltpu.VMEM((P_TILE,), jnp.float32),
            y_v=pltpu.VMEM((P_TILE,), jnp.float32),
            a_v=pltpu.VMEM((P_TILE,), jnp.float32),
            lw_v=pltpu.VMEM((P_TILE,), jnp.float32),
            obs_v=pltpu.VMEM((C,), jnp.float32),
            w_v=pltpu.VMEM((P_TILE,), jnp.float32),
            idx_refs=[pltpu.VMEM((2 * P_TILE,), jnp.int32) for _ in range(4)],
            wg_refs=[pltpu.VMEM((P_TILE,), jnp.float32) for _ in range(8)],
            rows_a=[pltpu.VMEM((2 * CH, C), jnp.float32) for _ in range(4)],
            rows_b=[pltpu.VMEM((2 * CH, C), jnp.float32) for _ in range(4)],
            sem_a=pltpu.SemaphoreType.DMA,
            sem_b=pltpu.SemaphoreType.DMA,
        ),
        compiler_params=pltpu.CompilerParams(needs_layout_passes=False,
                                             use_tc_tiling_on_sc=False),
    )
    return fn(xs, ys, angs, lws, obs_flat, table)


def _norm_body(w_ref, o_ref):
    w = w_ref[...]
    o_ref[...] = w / jnp.sum(w, axis=1, keepdims=True)


def kernel(particles, encoded_global_map, encoded_observations,
           unnormalized_resampled_particle_log_weights):
    table = jnp.moveaxis(encoded_global_map, 1, -1).reshape(B * H * W * R, C)
    w_un = _sc_weights(
        particles[..., 0].reshape(BN),
        particles[..., 1].reshape(BN),
        particles[..., 2].reshape(BN),
        unnormalized_resampled_particle_log_weights.reshape(BN),
        encoded_observations.reshape(B * C), table).reshape(B, N)
    return pl.pallas_call(
        _norm_body,
        out_shape=jax.ShapeDtypeStruct((B, N), jnp.float32),
    )(w_un)
